# Initial kernel scaffold; baseline (speedup 1.0000x reference)
import jax
import jax.numpy as jnp
from jax.experimental import pallas as pl


def kernel(position, heading, valid_mask, pos_table0, pos_table1, head_table0, head_table1, W0, b0, g0, be0, W1, b1, g1, be1, W2, b2, oob_w):
    raise NotImplementedError("write your pallas kernel here")


# speedup vs baseline: 4.2660x; 4.2660x over previous
import functools
import math

import jax
import jax.numpy as jnp
from jax import lax
from jax.experimental import pallas as pl
from jax.experimental.pallas import tpu as pltpu
from jax.experimental.pallas import tpu_sc as plsc

_PI = math.pi


def _vq_idx(v, d0, n0, d1, n1):
    i0 = jnp.clip((v / d0).astype(jnp.int32), 0, n0 - 1)
    r = v - i0.astype(jnp.float32) * d0
    i1 = jnp.clip((r / d1).astype(jnp.int32), 0, n1 - 1)
    return i0, i1


def _sc_feat_body(ntok, chunk, xs, ys, hs, t0, t1, h0, h1, feat_hbm,
                  xv, yv, hv, t0v, t1v, h0v, h1v, fb):
    wid = lax.axis_index("s") * 2 + lax.axis_index("c")
    base = wid * ntok
    pltpu.sync_copy(xs.at[pl.ds(base, ntok)], xv)
    pltpu.sync_copy(ys.at[pl.ds(base, ntok)], yv)
    pltpu.sync_copy(hs.at[pl.ds(base, ntok)], hv)
    pltpu.sync_copy(t0, t0v)
    pltpu.sync_copy(t1, t1v)
    pltpu.sync_copy(h0, h0v)
    pltpu.sync_copy(h1, h1v)

    zeros16 = jnp.zeros((16,), jnp.float32)

    def zero_body(i, carry):
        fb[pl.ds(i * 16, 16)] = zeros16
        return carry

    lax.fori_loop(0, chunk * 128 // 16, zero_body, None)

    iota = lax.iota(jnp.int32, 16)
    ngroups = chunk // 16
    nchunks = ntok // chunk

    def group_body(c, g, carry):
        off = c * chunk + g * 16
        x = xv[pl.ds(off, 16)]
        y = yv[pl.ds(off, 16)]
        h = hv[pl.ds(off, 16)]
        tx = x + 300.0
        ty = y + 300.0
        th = (h * 180.0) / _PI + 180.0
        ix0, ix1 = _vq_idx(tx, 1.0, 600, 0.01, 100)
        iy0, iy1 = _vq_idx(ty, 1.0, 600, 0.01, 100)
        ih0, ih1 = _vq_idx(th, 20.0, 20, 1.0, 20)
        row = (g * 16 + iota) * 128
        gx0 = ix0 * 24
        gx1 = ix1 * 24
        gy0 = iy0 * 24
        gy1 = iy1 * 24
        gh0 = ih0 * 6
        gh1 = ih1 * 6
        for j in range(24):
            plsc.store_scatter(fb, [row + j], plsc.load_gather(t0v, [gx0 + j]))
            plsc.store_scatter(fb, [row + (24 + j)],
                               plsc.load_gather(t1v, [gx1 + j]))
            plsc.store_scatter(fb, [row + (48 + j)],
                               plsc.load_gather(t0v, [gy0 + j]))
            plsc.store_scatter(fb, [row + (72 + j)],
                               plsc.load_gather(t1v, [gy1 + j]))
        for j in range(6):
            plsc.store_scatter(fb, [row + (96 + j)],
                               plsc.load_gather(h0v, [gh0 + j]))
            plsc.store_scatter(fb, [row + (102 + j)],
                               plsc.load_gather(h1v, [gh1 + j]))
        return carry

    def chunk_body(c, carry):
        lax.fori_loop(0, ngroups, functools.partial(group_body, c), None)
        pltpu.sync_copy(
            fb, feat_hbm.at[pl.ds((base + c * chunk) * 128, chunk * 128)])
        return carry

    lax.fori_loop(0, nchunks, chunk_body, None)


def _sc_feat(xs, ys, hs, t0f, t1f, h0f, h1f, n_tokens):
    nw = 32
    ntok = n_tokens // nw
    chunk = 512
    mesh = plsc.VectorSubcoreMesh(core_axis_name="c", subcore_axis_name="s")
    return pl.kernel(
        functools.partial(_sc_feat_body, ntok, chunk),
        out_type=jax.ShapeDtypeStruct((n_tokens * 128,), jnp.float32),
        mesh=mesh,
        compiler_params=pltpu.CompilerParams(needs_layout_passes=False),
        scratch_types=[
            pltpu.VMEM((ntok,), jnp.float32),
            pltpu.VMEM((ntok,), jnp.float32),
            pltpu.VMEM((ntok,), jnp.float32),
            pltpu.VMEM((600 * 24,), jnp.float32),
            pltpu.VMEM((100 * 24,), jnp.float32),
            pltpu.VMEM((20 * 6,), jnp.float32),
            pltpu.VMEM((20 * 6,), jnp.float32),
            pltpu.VMEM((chunk * 128,), jnp.float32),
        ],
    )(xs, ys, hs, t0f, t1f, h0f, h1f)


def _ln(x, g, b, eps=1e-5):
    mu = jnp.mean(x, axis=-1, keepdims=True)
    xc = x - mu
    var = jnp.mean(xc * xc, axis=-1, keepdims=True)
    return xc * lax.rsqrt(var + eps) * g + b


def _mlp_body(feat_ref, mask_ref, w0_ref, b0_ref, g0_ref, be0_ref,
              w1_ref, b1_ref, g1_ref, be1_ref, w2_ref, b2_ref, oob_ref,
              out_ref):
    f = feat_ref[...].astype(jnp.bfloat16)
    h = jnp.dot(f, w0_ref[...], preferred_element_type=jnp.float32)
    h = _ln(h + b0_ref[...], g0_ref[...], be0_ref[...])
    h = jnp.maximum(h, 0.0).astype(jnp.bfloat16)
    h = jnp.dot(h, w1_ref[...], preferred_element_type=jnp.float32)
    h = _ln(h + b1_ref[...], g1_ref[...], be1_ref[...])
    h = jnp.maximum(h, 0.0).astype(jnp.bfloat16)
    h = jnp.dot(h, w2_ref[...], preferred_element_type=jnp.float32)
    h = h + b2_ref[...]
    valid = mask_ref[...] != 0
    out_ref[...] = jnp.where(valid, h, oob_ref[...])


def _mlp(feat2d, mask2d, w0p, b0, g0, be0, w1, b1, g1, be1, w2, b2, oob,
         n_tokens, bm=512):
    full = lambda shape: pl.BlockSpec(shape, lambda i: (0, 0))
    return pl.pallas_call(
        _mlp_body,
        grid=(n_tokens // bm,),
        in_specs=[
            pl.BlockSpec((bm, 128), lambda i: (i, 0)),
            pl.BlockSpec((bm, 1), lambda i: (i, 0)),
            full((128, 256)),
            full((1, 256)), full((1, 256)), full((1, 256)),
            full((256, 256)),
            full((1, 256)), full((1, 256)), full((1, 256)),
            full((256, 256)),
            full((1, 256)), full((1, 256)),
        ],
        out_specs=pl.BlockSpec((bm, 256), lambda i: (i, 0)),
        out_shape=jax.ShapeDtypeStruct((n_tokens, 256), jnp.float32),
    )(feat2d, mask2d, w0p, b0, g0, be0, w1, b1, g1, be1, w2, b2, oob)


def kernel(position, heading, valid_mask, pos_table0, pos_table1,
           head_table0, head_table1, W0, b0, g0, be0, W1, b1, g1, be1,
           W2, b2, oob_w):
    B, A, T = heading.shape
    n = B * A * T
    xs = position[..., 0].reshape(n)
    ys = position[..., 1].reshape(n)
    hs = heading.reshape(n)
    mask2d = valid_mask.reshape(n, 1).astype(jnp.int32)

    feat = _sc_feat(xs, ys, hs, pos_table0.reshape(-1),
                    pos_table1.reshape(-1), head_table0.reshape(-1),
                    head_table1.reshape(-1), n).reshape(n, 128)

    w0p = jnp.zeros((128, 256), jnp.float32).at[:108].set(W0)
    row = lambda v: v.reshape(1, 256)
    out = _mlp(feat, mask2d,
               w0p.astype(jnp.bfloat16), row(b0), row(g0), row(be0),
               W1.astype(jnp.bfloat16), row(b1), row(g1), row(be1),
               W2.astype(jnp.bfloat16), row(b2), row(oob_w), n)
    return out.reshape(B, A, T, 256)
